# baseline (speedup 1.0000x reference)
import functools

import jax
import jax.numpy as jnp
from jax import lax
from jax.experimental import pallas as pl
from jax.experimental.pallas import tpu as pltpu
from jax.experimental.pallas import tpu_sc as plsc

NC, NS, LANES = 2, 16, 16
CH = 128
NBUF = 8
F = 16


def _make_sc_layer(n_rows, n_chunks_total, per_core_chunks, feature_split,
                   with_cnt):
  N = 100000
  per_tile = (per_core_chunks // NS) // NBUF * NBUF
  rem = per_core_chunks - per_tile * NS
  assert rem % NBUF == 0
  extra_tiles = rem // NBUF
  base_outer = per_tile // NBUF
  rows_per_tile = (N // NS) // 8 * 8
  rows_rem = N - rows_per_tile * NS

  mesh = plsc.VectorSubcoreMesh(core_axis_name="c", subcore_axis_name="s")

  @functools.partial(
      pl.kernel,
      out_type=jax.ShapeDtypeStruct((NC, N, F), jnp.float32),
      mesh=mesh,
      compiler_params=pltpu.CompilerParams(needs_layout_passes=False,
                                           use_tc_tiling_on_sc=False),
      scratch_types=[
          pltpu.VMEM_SHARED((N, F), jnp.float32),
          pltpu.VMEM((2, NBUF, 3, CH), jnp.int32),
          pltpu.VMEM((NBUF, CH), jnp.int32),
          pltpu.VMEM((NBUF, CH, F), jnp.float32),
          pltpu.VMEM((LANES,), jnp.float32),
          pltpu.SemaphoreType.DMA,
          pltpu.SemaphoreType.DMA((NBUF,)),
          pltpu.SemaphoreType.DMA((NBUF,)),
      ],
  )
  def sc_kernel(y_hbm, epk_hbm, bias_hbm, zeros_hbm, out_hbm,
                acc_sh, ebuf, ibuf, gbuf, bias_v, isem, gsem, ssem):
    c = lax.axis_index("c")
    s = lax.axis_index("s")

    lo = s * rows_per_tile
    pltpu.sync_copy(zeros_hbm.at[pl.ds(lo, rows_per_tile)],
                    acc_sh.at[pl.ds(lo, rows_per_tile)])

    @pl.when(s == NS - 1)
    def _():
      pltpu.sync_copy(zeros_hbm.at[pl.ds(rows_per_tile * NS, rows_rem)],
                      acc_sh.at[pl.ds(rows_per_tile * NS, rows_rem)])

    pltpu.sync_copy(bias_hbm.at[c], bias_v)
    plsc.subcore_barrier()

    core_base = jnp.int32(0) if feature_split else c * per_core_chunks
    base = core_base + s * per_tile + NBUF * jnp.minimum(s, extra_tiles)
    n_outer = base_outer + jnp.where(s < extra_tiles, 1, 0)
    hi_clamp = core_base + per_core_chunks - NBUF

    iota = lax.iota(jnp.int32, LANES)
    bias = bias_v[...]
    lanemask = iota < 8
    cvec = jnp.where(iota == 8, 1.0, 0.0).astype(jnp.float32)

    def fire_idx(g_next, pbuf):
      ch0 = jnp.minimum(base + g_next * NBUF, hi_clamp)
      pltpu.async_copy(epk_hbm.at[pl.ds(ch0, NBUF)], ebuf.at[pbuf], isem)

    fire_idx(0, 0)

    def outer(g, _):
      pg = lax.rem(g, 2)
      pltpu.make_async_copy(epk_hbm.at[pl.ds(0, NBUF)], ebuf.at[pg],
                            isem).wait()
      @pl.when(g > 0)
      def _():
        for b in range(NBUF):
          pltpu.make_async_copy(gbuf.at[b], acc_sh.at[pl.ds(0, CH)],
                                ssem.at[b]).wait()

      for b in range(NBUF):
        if feature_split:
          off = c * (n_rows // NC)
          for j in range(CH // LANES):
            v = ebuf[pg, b, 1, pl.ds(j * LANES, LANES)]
            ibuf[b, pl.ds(j * LANES, LANES)] = v + off
          idx_ref = ibuf.at[b]
        else:
          idx_ref = ebuf.at[pg, b, 1]
        pltpu.async_copy(y_hbm.at[idx_ref], gbuf.at[b], gsem.at[b])

      fire_idx(g + 1, 1 - pg)

      for b in range(NBUF):
        pltpu.make_async_copy(y_hbm.at[pl.ds(0, CH)], gbuf.at[b],
                              gsem.at[b]).wait()

        def edge_body(e, carry):
          es = jnp.full((LANES,), e, jnp.int32)
          wv = plsc.bitcast(plsc.load_gather(ebuf.at[pg, b, 2], [es]),
                            jnp.float32)
          gv = gbuf[b, e]
          t = wv * gv + bias
          r = jnp.maximum(t, 0.01 * t)
          if with_cnt:
            r = jnp.where(lanemask, r, cvec)
          gbuf[b, e] = r
          return carry

        lax.fori_loop(0, CH, edge_body, 0, unroll=16)
        pltpu.async_copy(gbuf.at[b], acc_sh.at[ebuf.at[pg, b, 0]],
                         ssem.at[b], add=True)
      return 0

    lax.fori_loop(0, n_outer, outer, 0)
    pltpu.make_async_copy(epk_hbm.at[pl.ds(0, NBUF)], ebuf.at[0], isem).wait()
    for b in range(NBUF):
      pltpu.make_async_copy(gbuf.at[b], acc_sh.at[pl.ds(0, CH)],
                            ssem.at[b]).wait()
    plsc.subcore_barrier()
    pltpu.sync_copy(acc_sh.at[pl.ds(lo, rows_per_tile)],
                    out_hbm.at[c, pl.ds(lo, rows_per_tile)])

    @pl.when(s == NS - 1)
    def _():
      pltpu.sync_copy(acc_sh.at[pl.ds(rows_per_tile * NS, rows_rem)],
                      out_hbm.at[c, pl.ds(rows_per_tile * NS, rows_rem)])

  return sc_kernel


def _lrelu(t):
  return jnp.where(t >= 0, t, 0.01 * t)


def _dot_t(a, w):
  return lax.dot_general(a, w, (((1,), (1,)), ((), ())),
                         preferred_element_type=jnp.float32)


def _tc_y1(x, W1, N, R):
  def body(x_ref, w_ref, o_ref):
    y = _dot_t(x_ref[...], w_ref[...])
    o_ref[...] = jnp.concatenate(
        [y, jnp.zeros((R, 16 - y.shape[1]), jnp.float32)], axis=1)

  return pl.pallas_call(
      body,
      grid=(N // R,),
      in_specs=[pl.BlockSpec((R, 4), lambda i: (i, 0)),
                pl.BlockSpec((8, 4), lambda i: (0, 0))],
      out_specs=pl.BlockSpec((R, 16), lambda i: (i, 0)),
      out_shape=jax.ShapeDtypeStruct((N, 16), jnp.float32),
  )(x, W1)


def _tc_combine1(acc1, y1p, b1, W3, N, R):
  def body(a_ref, y_ref, b_ref, w_ref, y2_ref, ic_ref):
    A = a_ref[0] + a_ref[1]
    cnt = A[:, 8:9]
    invc = 1.0 / jnp.maximum(cnt, 1.0)
    out1 = A[:, :8] * invc + _lrelu(y_ref[...][:, :8] + b_ref[...])
    y2_ref[...] = _dot_t(out1, w_ref[...])
    ic_ref[...] = jnp.broadcast_to(invc, (R, 8))

  return pl.pallas_call(
      body,
      grid=(N // R,),
      in_specs=[pl.BlockSpec((2, R, 16), lambda i: (0, i, 0)),
                pl.BlockSpec((R, 16), lambda i: (i, 0)),
                pl.BlockSpec((1, 8), lambda i: (0, 0)),
                pl.BlockSpec((16, 8), lambda i: (0, 0))],
      out_specs=[pl.BlockSpec((R, 16), lambda i: (i, 0)),
                 pl.BlockSpec((R, 8), lambda i: (i, 0))],
      out_shape=[jax.ShapeDtypeStruct((N, 16), jnp.float32),
                 jax.ShapeDtypeStruct((N, 8), jnp.float32)],
  )(acc1, y1p, b1, W3)


def _tc_combine2(acc2, y2, invc, b3, W5, N, R):
  def body(a_ref, y_ref, ic_ref, b_ref, w_ref, y3_ref):
    A = a_ref[0] + a_ref[1]
    invc = ic_ref[...][:, :1]
    out2 = A * invc + _lrelu(y_ref[...] + b_ref[...])
    y3 = _dot_t(out2, w_ref[...])
    y3_ref[0] = y3[:, :16]
    y3_ref[1] = y3[:, 16:]

  return pl.pallas_call(
      body,
      grid=(N // R,),
      in_specs=[pl.BlockSpec((2, R, 16), lambda i: (0, i, 0)),
                pl.BlockSpec((R, 16), lambda i: (i, 0)),
                pl.BlockSpec((R, 8), lambda i: (i, 0)),
                pl.BlockSpec((1, 16), lambda i: (0, 0)),
                pl.BlockSpec((32, 16), lambda i: (0, 0))],
      out_specs=pl.BlockSpec((2, R, 16), lambda i: (0, i, 0)),
      out_shape=jax.ShapeDtypeStruct((2, N, 16), jnp.float32),
  )(acc2, y2, invc, b3, W5)


def _tc_final(acc3, y3, invc, b5, W7, b7, N, R):
  ngrid = N // R

  def body(a_ref, y_ref, ic_ref, b_ref, w_ref, b7_ref, o_ref, psum):
    i = pl.program_id(0)
    A = jnp.concatenate([a_ref[0], a_ref[1]], axis=1)
    Y = jnp.concatenate([y_ref[0], y_ref[1]], axis=1)
    out3 = A * ic_ref[...][:, :1] + _lrelu(Y + b_ref[...])
    blksum = jnp.sum(out3, axis=0, keepdims=True)

    @pl.when(i == 0)
    def _():
      psum[...] = jnp.zeros_like(psum)

    psum[...] += blksum

    @pl.when(i == ngrid - 1)
    def _():
      pooled = psum[...] * jnp.float32(1.0 / N)
      z = _dot_t(pooled, w_ref[...]) + b7_ref[...]
      m = jnp.max(z, axis=1, keepdims=True)
      zz = z - m
      o_ref[...] = zz - jnp.log(jnp.sum(jnp.exp(zz), axis=1, keepdims=True))

  return pl.pallas_call(
      body,
      grid=(ngrid,),
      in_specs=[pl.BlockSpec((2, R, 16), lambda i: (0, i, 0)),
                pl.BlockSpec((2, R, 16), lambda i: (0, i, 0)),
                pl.BlockSpec((R, 8), lambda i: (i, 0)),
                pl.BlockSpec((1, 32), lambda i: (0, 0)),
                pl.BlockSpec((2, 32), lambda i: (0, 0)),
                pl.BlockSpec((1, 2), lambda i: (0, 0))],
      out_specs=pl.BlockSpec((1, 2), lambda i: (0, 0)),
      out_shape=jax.ShapeDtypeStruct((1, 2), jnp.float32),
      scratch_shapes=[pltpu.VMEM((1, 32), jnp.float32)],
  )(acc3, y3, invc, b5, W7, b7)


@jax.jit
def kernel(x, edge, weight, W1, b1, W3, b3, W5, b5, W7, b7):
  N = x.shape[0]
  E = edge.shape[1]
  R = 2000
  n_chunks = E // CH

  edge32 = edge.astype(jnp.int32)
  wbits = lax.bitcast_convert_type(weight.astype(jnp.float32), jnp.int32)
  epk = jnp.stack([edge32[0].reshape(n_chunks, CH),
                   edge32[1].reshape(n_chunks, CH),
                   wbits.reshape(n_chunks, CH)], axis=1)
  zeros16 = jnp.zeros((N, F), jnp.float32)

  sc1 = _make_sc_layer(N, n_chunks, n_chunks // NC, False, True)
  sc2 = _make_sc_layer(N, n_chunks, n_chunks // NC, False, False)
  sc3 = _make_sc_layer(2 * N, n_chunks, n_chunks, True, False)

  b1e = jnp.concatenate([b1, jnp.zeros((8,), jnp.float32)])
  bias1 = jnp.tile(b1e.reshape(1, 16), (NC, 1))
  bias2 = jnp.tile(b3.reshape(1, 16), (NC, 1))
  bias3 = b5.reshape(NC, 16)

  y1p = _tc_y1(x, W1, N, R)
  acc1 = sc1(y1p, epk, bias1, zeros16)
  y2, invc = _tc_combine1(acc1, y1p, b1.reshape(1, 8), W3, N, R)
  acc2 = sc2(y2, epk, bias2, zeros16)
  y3 = _tc_combine2(acc2, y2, invc, b3.reshape(1, 16), W5, N, R)
  acc3 = sc3(y3.reshape(2 * N, 16), epk, bias3, zeros16)
  return _tc_final(acc3, y3, invc, b5.reshape(1, 32), W7,
                   b7.reshape(1, 2), N, R)


# speedup vs baseline: 2.8827x; 2.8827x over previous
import functools

import jax
import jax.numpy as jnp
from jax import lax
from jax.experimental import pallas as pl
from jax.experimental.pallas import tpu as pltpu
from jax.experimental.pallas import tpu_sc as plsc

NC, NS, LANES = 2, 16, 16
CH = 128
NBUF = 4
F = 16


def _make_sc_layer(n_rows, n_chunks_total, per_core_chunks, feature_split,
                   with_cnt):
  N = 100000
  per_tile = (per_core_chunks // NS) // NBUF * NBUF
  rem = per_core_chunks - per_tile * NS
  assert rem % NBUF == 0
  extra_tiles = rem // NBUF
  base_outer = per_tile // NBUF
  rows_per_tile = (N // NS) // 8 * 8
  rows_rem = N - rows_per_tile * NS

  mesh = plsc.VectorSubcoreMesh(core_axis_name="c", subcore_axis_name="s")

  @functools.partial(
      pl.kernel,
      out_type=jax.ShapeDtypeStruct((NC, N, F), jnp.float32),
      mesh=mesh,
      compiler_params=pltpu.CompilerParams(needs_layout_passes=False,
                                           use_tc_tiling_on_sc=False),
      scratch_types=[
          pltpu.VMEM_SHARED((N, F), jnp.float32),
          pltpu.VMEM((2, NBUF, 3, CH), jnp.int32),
          pltpu.VMEM((NBUF, CH), jnp.int32),
          pltpu.VMEM((NBUF, CH, F), jnp.float32),
          pltpu.VMEM((LANES,), jnp.float32),
          pltpu.SemaphoreType.DMA,
          pltpu.SemaphoreType.DMA((NBUF,)),
          pltpu.SemaphoreType.DMA((NBUF,)),
      ],
  )
  def sc_kernel(y_hbm, epk_hbm, bias_hbm, zeros_hbm, out_hbm,
                acc_sh, ebuf, ibuf, gbuf, bias_v, isem, gsem, ssem):
    c = lax.axis_index("c")
    s = lax.axis_index("s")

    lo = s * rows_per_tile
    pltpu.sync_copy(zeros_hbm.at[pl.ds(lo, rows_per_tile)],
                    acc_sh.at[pl.ds(lo, rows_per_tile)])

    @pl.when(s == NS - 1)
    def _():
      pltpu.sync_copy(zeros_hbm.at[pl.ds(rows_per_tile * NS, rows_rem)],
                      acc_sh.at[pl.ds(rows_per_tile * NS, rows_rem)])

    pltpu.sync_copy(bias_hbm.at[c], bias_v)
    plsc.subcore_barrier()

    core_base = jnp.int32(0) if feature_split else c * per_core_chunks
    base = core_base + s * per_tile + NBUF * jnp.minimum(s, extra_tiles)
    n_outer = base_outer + jnp.where(s < extra_tiles, 1, 0)
    hi_clamp = core_base + per_core_chunks - NBUF

    iota = lax.iota(jnp.int32, LANES)
    bias = bias_v[...]
    lanemask = iota < 8
    cvec = jnp.where(iota == 8, 1.0, 0.0).astype(jnp.float32)

    def fire_idx(g_next, pbuf):
      ch0 = jnp.minimum(base + g_next * NBUF, hi_clamp)
      pltpu.async_copy(epk_hbm.at[pl.ds(ch0, NBUF)], ebuf.at[pbuf], isem)

    fire_idx(0, 0)

    def outer(g, _):
      pg = lax.rem(g, 2)
      pltpu.make_async_copy(epk_hbm.at[pl.ds(0, NBUF)], ebuf.at[pg],
                            isem).wait()
      @pl.when(g > 0)
      def _():
        for b in range(NBUF):
          pltpu.make_async_copy(gbuf.at[b], acc_sh.at[pl.ds(0, CH)],
                                ssem.at[b]).wait()

      for b in range(NBUF):
        if feature_split:
          off = c * (n_rows // NC)
          for j in range(CH // LANES):
            v = ebuf[pg, b, 1, pl.ds(j * LANES, LANES)]
            ibuf[b, pl.ds(j * LANES, LANES)] = v + off
          idx_ref = ibuf.at[b]
        else:
          idx_ref = ebuf.at[pg, b, 1]
        pltpu.async_copy(y_hbm.at[idx_ref], gbuf.at[b], gsem.at[b])

      fire_idx(g + 1, 1 - pg)

      for b in range(NBUF):
        pltpu.make_async_copy(y_hbm.at[pl.ds(0, CH)], gbuf.at[b],
                              gsem.at[b]).wait()

        for k in range(CH // LANES):
          wvec = plsc.bitcast(ebuf[pg, b, 2, pl.ds(k * LANES, LANES)],
                              jnp.float32)
          for j in range(LANES):
            e = k * LANES + j
            wj = wvec.at[jnp.full((LANES,), j, jnp.int32)].get(
                mode="promise_in_bounds")
            t = wj * gbuf[b, e] + bias
            r = jnp.maximum(t, 0.01 * t)
            if with_cnt:
              r = jnp.where(lanemask, r, cvec)
            gbuf[b, e] = r

        pltpu.async_copy(gbuf.at[b], acc_sh.at[ebuf.at[pg, b, 0]],
                         ssem.at[b], add=True)
      return 0

    lax.fori_loop(0, n_outer, outer, 0)
    pltpu.make_async_copy(epk_hbm.at[pl.ds(0, NBUF)], ebuf.at[0], isem).wait()
    for b in range(NBUF):
      pltpu.make_async_copy(gbuf.at[b], acc_sh.at[pl.ds(0, CH)],
                            ssem.at[b]).wait()
    plsc.subcore_barrier()
    pltpu.sync_copy(acc_sh.at[pl.ds(lo, rows_per_tile)],
                    out_hbm.at[c, pl.ds(lo, rows_per_tile)])

    @pl.when(s == NS - 1)
    def _():
      pltpu.sync_copy(acc_sh.at[pl.ds(rows_per_tile * NS, rows_rem)],
                      out_hbm.at[c, pl.ds(rows_per_tile * NS, rows_rem)])

  return sc_kernel


def _lrelu(t):
  return jnp.where(t >= 0, t, 0.01 * t)


def _dot_t(a, w):
  return lax.dot_general(a, w, (((1,), (1,)), ((), ())),
                         preferred_element_type=jnp.float32)


def _tc_y1(x, W1, N, R):
  def body(x_ref, w_ref, o_ref):
    y = _dot_t(x_ref[...], w_ref[...])
    o_ref[...] = jnp.concatenate(
        [y, jnp.zeros((R, 16 - y.shape[1]), jnp.float32)], axis=1)

  return pl.pallas_call(
      body,
      grid=(N // R,),
      in_specs=[pl.BlockSpec((R, 4), lambda i: (i, 0)),
                pl.BlockSpec((8, 4), lambda i: (0, 0))],
      out_specs=pl.BlockSpec((R, 16), lambda i: (i, 0)),
      out_shape=jax.ShapeDtypeStruct((N, 16), jnp.float32),
  )(x, W1)


def _tc_combine1(acc1, y1p, b1, W3, N, R):
  def body(a_ref, y_ref, b_ref, w_ref, y2_ref, ic_ref):
    A = a_ref[0] + a_ref[1]
    cnt = A[:, 8:9]
    invc = 1.0 / jnp.maximum(cnt, 1.0)
    out1 = A[:, :8] * invc + _lrelu(y_ref[...][:, :8] + b_ref[...])
    y2_ref[...] = _dot_t(out1, w_ref[...])
    ic_ref[...] = jnp.broadcast_to(invc, (R, 8))

  return pl.pallas_call(
      body,
      grid=(N // R,),
      in_specs=[pl.BlockSpec((2, R, 16), lambda i: (0, i, 0)),
                pl.BlockSpec((R, 16), lambda i: (i, 0)),
                pl.BlockSpec((1, 8), lambda i: (0, 0)),
                pl.BlockSpec((16, 8), lambda i: (0, 0))],
      out_specs=[pl.BlockSpec((R, 16), lambda i: (i, 0)),
                 pl.BlockSpec((R, 8), lambda i: (i, 0))],
      out_shape=[jax.ShapeDtypeStruct((N, 16), jnp.float32),
                 jax.ShapeDtypeStruct((N, 8), jnp.float32)],
  )(acc1, y1p, b1, W3)


def _tc_combine2(acc2, y2, invc, b3, W5, N, R):
  def body(a_ref, y_ref, ic_ref, b_ref, w_ref, y3_ref):
    A = a_ref[0] + a_ref[1]
    invc = ic_ref[...][:, :1]
    out2 = A * invc + _lrelu(y_ref[...] + b_ref[...])
    y3 = _dot_t(out2, w_ref[...])
    y3_ref[0] = y3[:, :16]
    y3_ref[1] = y3[:, 16:]

  return pl.pallas_call(
      body,
      grid=(N // R,),
      in_specs=[pl.BlockSpec((2, R, 16), lambda i: (0, i, 0)),
                pl.BlockSpec((R, 16), lambda i: (i, 0)),
                pl.BlockSpec((R, 8), lambda i: (i, 0)),
                pl.BlockSpec((1, 16), lambda i: (0, 0)),
                pl.BlockSpec((32, 16), lambda i: (0, 0))],
      out_specs=pl.BlockSpec((2, R, 16), lambda i: (0, i, 0)),
      out_shape=jax.ShapeDtypeStruct((2, N, 16), jnp.float32),
  )(acc2, y2, invc, b3, W5)


def _tc_final(acc3, y3, invc, b5, W7, b7, N, R):
  ngrid = N // R

  def body(a_ref, y_ref, ic_ref, b_ref, w_ref, b7_ref, o_ref, psum):
    i = pl.program_id(0)
    A = jnp.concatenate([a_ref[0], a_ref[1]], axis=1)
    Y = jnp.concatenate([y_ref[0], y_ref[1]], axis=1)
    out3 = A * ic_ref[...][:, :1] + _lrelu(Y + b_ref[...])
    blksum = jnp.sum(out3, axis=0, keepdims=True)

    @pl.when(i == 0)
    def _():
      psum[...] = jnp.zeros_like(psum)

    psum[...] += blksum

    @pl.when(i == ngrid - 1)
    def _():
      pooled = psum[...] * jnp.float32(1.0 / N)
      z = _dot_t(pooled, w_ref[...]) + b7_ref[...]
      m = jnp.max(z, axis=1, keepdims=True)
      zz = z - m
      o_ref[...] = zz - jnp.log(jnp.sum(jnp.exp(zz), axis=1, keepdims=True))

  return pl.pallas_call(
      body,
      grid=(ngrid,),
      in_specs=[pl.BlockSpec((2, R, 16), lambda i: (0, i, 0)),
                pl.BlockSpec((2, R, 16), lambda i: (0, i, 0)),
                pl.BlockSpec((R, 8), lambda i: (i, 0)),
                pl.BlockSpec((1, 32), lambda i: (0, 0)),
                pl.BlockSpec((2, 32), lambda i: (0, 0)),
                pl.BlockSpec((1, 2), lambda i: (0, 0))],
      out_specs=pl.BlockSpec((1, 2), lambda i: (0, 0)),
      out_shape=jax.ShapeDtypeStruct((1, 2), jnp.float32),
      scratch_shapes=[pltpu.VMEM((1, 32), jnp.float32)],
  )(acc3, y3, invc, b5, W7, b7)


@jax.jit
def kernel(x, edge, weight, W1, b1, W3, b3, W5, b5, W7, b7):
  N = x.shape[0]
  E = edge.shape[1]
  R = 2000
  n_chunks = E // CH

  edge32 = edge.astype(jnp.int32)
  wbits = lax.bitcast_convert_type(weight.astype(jnp.float32), jnp.int32)
  epk = jnp.stack([edge32[0].reshape(n_chunks, CH),
                   edge32[1].reshape(n_chunks, CH),
                   wbits.reshape(n_chunks, CH)], axis=1)
  zeros16 = jnp.zeros((N, F), jnp.float32)

  sc1 = _make_sc_layer(N, n_chunks, n_chunks // NC, False, True)
  sc2 = _make_sc_layer(N, n_chunks, n_chunks // NC, False, False)
  sc3 = _make_sc_layer(2 * N, n_chunks, n_chunks, True, False)

  b1e = jnp.concatenate([b1, jnp.zeros((8,), jnp.float32)])
  bias1 = jnp.tile(b1e.reshape(1, 16), (NC, 1))
  bias2 = jnp.tile(b3.reshape(1, 16), (NC, 1))
  bias3 = b5.reshape(NC, 16)

  y1p = _tc_y1(x, W1, N, R)
  acc1 = sc1(y1p, epk, bias1, zeros16)
  y2, invc = _tc_combine1(acc1, y1p, b1.reshape(1, 8), W3, N, R)
  acc2 = sc2(y2, epk, bias2, zeros16)
  y3 = _tc_combine2(acc2, y2, invc, b3.reshape(1, 16), W5, N, R)
  acc3 = sc3(y3.reshape(2 * N, 16), epk, bias3, zeros16)
  return _tc_final(acc3, y3, invc, b5.reshape(1, 32), W7,
                   b7.reshape(1, 2), N, R)


# speedup vs baseline: 3.0476x; 1.0572x over previous
import functools

import jax
import jax.numpy as jnp
from jax import lax
from jax.experimental import pallas as pl
from jax.experimental.pallas import tpu as pltpu
from jax.experimental.pallas import tpu_sc as plsc

NC, NS, LANES = 2, 16, 16
CH = 128
NBUF = 4
F = 16


def _make_sc_layer(n_rows, n_chunks_total, per_core_chunks, feature_split,
                   with_cnt):
  N = 100000
  per_tile = (per_core_chunks // NS) // NBUF * NBUF
  rem = per_core_chunks - per_tile * NS
  assert rem % NBUF == 0
  extra_tiles = rem // NBUF
  base_outer = per_tile // NBUF
  rows_per_tile = (N // NS) // 8 * 8
  rows_rem = N - rows_per_tile * NS

  mesh = plsc.VectorSubcoreMesh(core_axis_name="c", subcore_axis_name="s")

  @functools.partial(
      pl.kernel,
      out_type=jax.ShapeDtypeStruct((NC, N, F), jnp.float32),
      mesh=mesh,
      compiler_params=pltpu.CompilerParams(needs_layout_passes=False,
                                           use_tc_tiling_on_sc=False),
      scratch_types=[
          pltpu.VMEM_SHARED((N, F), jnp.float32),
          pltpu.VMEM((2, NBUF, CH), jnp.int32),
          pltpu.VMEM((2, NBUF, CH), jnp.int32),
          pltpu.VMEM((2, NBUF, CH), jnp.float32),
          pltpu.VMEM((NBUF, CH), jnp.int32),
          pltpu.VMEM((NBUF, CH, F), jnp.float32),
          pltpu.VMEM((LANES,), jnp.float32),
          pltpu.SemaphoreType.DMA,
          pltpu.SemaphoreType.DMA((NBUF,)),
          pltpu.SemaphoreType.DMA((NBUF,)),
      ],
  )
  def sc_kernel(y_hbm, rows_hbm, cols_hbm, w_hbm, bias_hbm, zeros_hbm,
                out_hbm, acc_sh, rbuf, cbuf, wbuf, ibuf, gbuf, bias_v,
                isem, gsem, ssem):
    c = lax.axis_index("c")
    s = lax.axis_index("s")

    lo = s * rows_per_tile
    pltpu.sync_copy(zeros_hbm.at[pl.ds(lo, rows_per_tile)],
                    acc_sh.at[pl.ds(lo, rows_per_tile)])

    @pl.when(s == NS - 1)
    def _():
      pltpu.sync_copy(zeros_hbm.at[pl.ds(rows_per_tile * NS, rows_rem)],
                      acc_sh.at[pl.ds(rows_per_tile * NS, rows_rem)])

    pltpu.sync_copy(bias_hbm.at[c], bias_v)
    plsc.subcore_barrier()

    core_base = jnp.int32(0) if feature_split else c * per_core_chunks
    base = core_base + s * per_tile + NBUF * jnp.minimum(s, extra_tiles)
    n_outer = base_outer + jnp.where(s < extra_tiles, 1, 0)
    hi_clamp = core_base + per_core_chunks - NBUF

    iota = lax.iota(jnp.int32, LANES)
    bias = bias_v[...]
    lanemask = iota < 8
    cvec = jnp.where(iota == 8, 1.0, 0.0).astype(jnp.float32)

    def fire_idx(g_next, pbuf):
      ch0 = jnp.minimum(base + g_next * NBUF, hi_clamp)
      pltpu.async_copy(rows_hbm.at[pl.ds(ch0, NBUF)], rbuf.at[pbuf], isem)
      pltpu.async_copy(cols_hbm.at[pl.ds(ch0, NBUF)], cbuf.at[pbuf], isem)
      pltpu.async_copy(w_hbm.at[pl.ds(ch0, NBUF)], wbuf.at[pbuf], isem)

    def drain_idx(pbuf):
      pltpu.make_async_copy(rows_hbm.at[pl.ds(0, NBUF)], rbuf.at[pbuf],
                            isem).wait()
      pltpu.make_async_copy(cols_hbm.at[pl.ds(0, NBUF)], cbuf.at[pbuf],
                            isem).wait()
      pltpu.make_async_copy(w_hbm.at[pl.ds(0, NBUF)], wbuf.at[pbuf],
                            isem).wait()

    fire_idx(0, 0)

    def outer(g, _):
      pg = lax.rem(g, 2)
      drain_idx(pg)
      @pl.when(g > 0)
      def _():
        for b in range(NBUF):
          pltpu.make_async_copy(gbuf.at[b], acc_sh.at[pl.ds(0, CH)],
                                ssem.at[b]).wait()

      for b in range(NBUF):
        if feature_split:
          off = c * (n_rows // NC)
          for j in range(CH // LANES):
            v = cbuf[pg, b, pl.ds(j * LANES, LANES)]
            ibuf[b, pl.ds(j * LANES, LANES)] = v + off
          idx_ref = ibuf.at[b]
        else:
          idx_ref = cbuf.at[pg, b]
        pltpu.async_copy(y_hbm.at[idx_ref], gbuf.at[b], gsem.at[b])

      fire_idx(g + 1, 1 - pg)

      for b in range(NBUF):
        pltpu.make_async_copy(y_hbm.at[pl.ds(0, CH)], gbuf.at[b],
                              gsem.at[b]).wait()

        for k in range(CH // LANES):
          wvec = wbuf[pg, b, pl.ds(k * LANES, LANES)]
          for j in range(LANES):
            e = k * LANES + j
            wj = wvec.at[jnp.full((LANES,), j, jnp.int32)].get(
                mode="promise_in_bounds")
            t = wj * gbuf[b, e] + bias
            r = jnp.maximum(t, 0.01 * t)
            if with_cnt:
              r = jnp.where(lanemask, r, cvec)
            gbuf[b, e] = r

        pltpu.async_copy(gbuf.at[b], acc_sh.at[rbuf.at[pg, b]],
                         ssem.at[b], add=True)
      return 0

    lax.fori_loop(0, n_outer, outer, 0)
    drain_idx(0)
    for b in range(NBUF):
      pltpu.make_async_copy(gbuf.at[b], acc_sh.at[pl.ds(0, CH)],
                            ssem.at[b]).wait()
    plsc.subcore_barrier()
    pltpu.sync_copy(acc_sh.at[pl.ds(lo, rows_per_tile)],
                    out_hbm.at[c, pl.ds(lo, rows_per_tile)])

    @pl.when(s == NS - 1)
    def _():
      pltpu.sync_copy(acc_sh.at[pl.ds(rows_per_tile * NS, rows_rem)],
                      out_hbm.at[c, pl.ds(rows_per_tile * NS, rows_rem)])

  return sc_kernel


def _lrelu(t):
  return jnp.where(t >= 0, t, 0.01 * t)


def _dot_t(a, w):
  return lax.dot_general(a, w, (((1,), (1,)), ((), ())),
                         preferred_element_type=jnp.float32)


def _tc_y1(x, W1, N, R):
  def body(x_ref, w_ref, o_ref):
    y = _dot_t(x_ref[...], w_ref[...])
    o_ref[...] = jnp.concatenate(
        [y, jnp.zeros((R, 16 - y.shape[1]), jnp.float32)], axis=1)

  return pl.pallas_call(
      body,
      grid=(N // R,),
      in_specs=[pl.BlockSpec((R, 4), lambda i: (i, 0)),
                pl.BlockSpec((8, 4), lambda i: (0, 0))],
      out_specs=pl.BlockSpec((R, 16), lambda i: (i, 0)),
      out_shape=jax.ShapeDtypeStruct((N, 16), jnp.float32),
  )(x, W1)


def _tc_combine1(acc1, y1p, b1, W3, N, R):
  def body(a_ref, y_ref, b_ref, w_ref, y2_ref, ic_ref):
    A = a_ref[0] + a_ref[1]
    cnt = A[:, 8:9]
    invc = 1.0 / jnp.maximum(cnt, 1.0)
    out1 = A[:, :8] * invc + _lrelu(y_ref[...][:, :8] + b_ref[...])
    y2_ref[...] = _dot_t(out1, w_ref[...])
    ic_ref[...] = jnp.broadcast_to(invc, (R, 8))

  return pl.pallas_call(
      body,
      grid=(N // R,),
      in_specs=[pl.BlockSpec((2, R, 16), lambda i: (0, i, 0)),
                pl.BlockSpec((R, 16), lambda i: (i, 0)),
                pl.BlockSpec((1, 8), lambda i: (0, 0)),
                pl.BlockSpec((16, 8), lambda i: (0, 0))],
      out_specs=[pl.BlockSpec((R, 16), lambda i: (i, 0)),
                 pl.BlockSpec((R, 8), lambda i: (i, 0))],
      out_shape=[jax.ShapeDtypeStruct((N, 16), jnp.float32),
                 jax.ShapeDtypeStruct((N, 8), jnp.float32)],
  )(acc1, y1p, b1, W3)


def _tc_combine2(acc2, y2, invc, b3, W5, N, R):
  def body(a_ref, y_ref, ic_ref, b_ref, w_ref, y3_ref):
    A = a_ref[0] + a_ref[1]
    invc = ic_ref[...][:, :1]
    out2 = A * invc + _lrelu(y_ref[...] + b_ref[...])
    y3 = _dot_t(out2, w_ref[...])
    y3_ref[0] = y3[:, :16]
    y3_ref[1] = y3[:, 16:]

  return pl.pallas_call(
      body,
      grid=(N // R,),
      in_specs=[pl.BlockSpec((2, R, 16), lambda i: (0, i, 0)),
                pl.BlockSpec((R, 16), lambda i: (i, 0)),
                pl.BlockSpec((R, 8), lambda i: (i, 0)),
                pl.BlockSpec((1, 16), lambda i: (0, 0)),
                pl.BlockSpec((32, 16), lambda i: (0, 0))],
      out_specs=pl.BlockSpec((2, R, 16), lambda i: (0, i, 0)),
      out_shape=jax.ShapeDtypeStruct((2, N, 16), jnp.float32),
  )(acc2, y2, invc, b3, W5)


def _tc_final(acc3, y3, invc, b5, W7, b7, N, R):
  ngrid = N // R

  def body(a_ref, y_ref, ic_ref, b_ref, w_ref, b7_ref, o_ref, psum):
    i = pl.program_id(0)
    A = jnp.concatenate([a_ref[0], a_ref[1]], axis=1)
    Y = jnp.concatenate([y_ref[0], y_ref[1]], axis=1)
    out3 = A * ic_ref[...][:, :1] + _lrelu(Y + b_ref[...])
    blksum = jnp.sum(out3, axis=0, keepdims=True)

    @pl.when(i == 0)
    def _():
      psum[...] = jnp.zeros_like(psum)

    psum[...] += blksum

    @pl.when(i == ngrid - 1)
    def _():
      pooled = psum[...] * jnp.float32(1.0 / N)
      z = _dot_t(pooled, w_ref[...]) + b7_ref[...]
      m = jnp.max(z, axis=1, keepdims=True)
      zz = z - m
      o_ref[...] = zz - jnp.log(jnp.sum(jnp.exp(zz), axis=1, keepdims=True))

  return pl.pallas_call(
      body,
      grid=(ngrid,),
      in_specs=[pl.BlockSpec((2, R, 16), lambda i: (0, i, 0)),
                pl.BlockSpec((2, R, 16), lambda i: (0, i, 0)),
                pl.BlockSpec((R, 8), lambda i: (i, 0)),
                pl.BlockSpec((1, 32), lambda i: (0, 0)),
                pl.BlockSpec((2, 32), lambda i: (0, 0)),
                pl.BlockSpec((1, 2), lambda i: (0, 0))],
      out_specs=pl.BlockSpec((1, 2), lambda i: (0, 0)),
      out_shape=jax.ShapeDtypeStruct((1, 2), jnp.float32),
      scratch_shapes=[pltpu.VMEM((1, 32), jnp.float32)],
  )(acc3, y3, invc, b5, W7, b7)


@jax.jit
def kernel(x, edge, weight, W1, b1, W3, b3, W5, b5, W7, b7):
  N = x.shape[0]
  E = edge.shape[1]
  R = 5000
  n_chunks = E // CH

  edge32 = edge.astype(jnp.int32)
  rows2d = edge32[0].reshape(n_chunks, CH)
  cols2d = edge32[1].reshape(n_chunks, CH)
  w2d = weight.astype(jnp.float32).reshape(n_chunks, CH)
  zeros16 = jnp.zeros((N, F), jnp.float32)

  sc1 = _make_sc_layer(N, n_chunks, n_chunks // NC, False, True)
  sc2 = _make_sc_layer(N, n_chunks, n_chunks // NC, False, False)
  sc3 = _make_sc_layer(2 * N, n_chunks, n_chunks, True, False)

  b1e = jnp.concatenate([b1, jnp.zeros((8,), jnp.float32)])
  bias1 = jnp.tile(b1e.reshape(1, 16), (NC, 1))
  bias2 = jnp.tile(b3.reshape(1, 16), (NC, 1))
  bias3 = b5.reshape(NC, 16)

  y1p = _tc_y1(x, W1, N, R)
  acc1 = sc1(y1p, rows2d, cols2d, w2d, bias1, zeros16)
  y2, invc = _tc_combine1(acc1, y1p, b1.reshape(1, 8), W3, N, R)
  acc2 = sc2(y2, rows2d, cols2d, w2d, bias2, zeros16)
  y3 = _tc_combine2(acc2, y2, invc, b3.reshape(1, 16), W5, N, R)
  acc3 = sc3(y3.reshape(2 * N, 16), rows2d, cols2d, w2d, bias3, zeros16)
  return _tc_final(acc3, y3, invc, b5.reshape(1, 32), W7,
                   b7.reshape(1, 2), N, R)
